# baseline (speedup 1.0000x reference)
import functools

import jax
import jax.numpy as jnp
from jax.experimental import pallas as pl
from jax.experimental.pallas import tpu as pltpu

B, L, H = 16, 4096, 128
LBLK = 512
NCHUNK = L // LBLK
BIG_IDX = 2**30


def _net_tree_kernel(stim_ref, vals_hbm, lens_ref, wk_ref, bk_ref, wv_ref,
                     bv_ref, x_ref, idx_ref, buf_ref, sems):

    def chunk_copy(i):
        return pltpu.make_async_copy(
            vals_hbm.at[:, pl.ds(i * LBLK, LBLK), :],
            buf_ref.at[i],
            sems.at[i])

    for i in range(NCHUNK):
        chunk_copy(i).start()

    k = jax.nn.relu(
        jnp.dot(stim_ref[...], wk_ref[...],
                preferred_element_type=jnp.float32) + bk_ref[...])
    wv = wv_ref[...]
    bv = bv_ref[...]
    lens = lens_ref[...]

    rmax = jnp.full((B, 128), -jnp.inf, dtype=jnp.float32)
    ridx = jnp.zeros((B, 128), dtype=jnp.int32)

    for i in range(NCHUNK):
        chunk_copy(i).wait()
        v = buf_ref[i].reshape(B * LBLK, H)

        pv = jax.nn.relu(
            jnp.dot(v, wv, preferred_element_type=jnp.float32) + bv)

        x = jnp.sum(pv.reshape(B, LBLK, H) * k[:, None, :], axis=-1)
        x_ref[:, i * LBLK:(i + 1) * LBLK] = x

        pos = jax.lax.broadcasted_iota(jnp.int32, (B, LBLK), 1) + i * LBLK
        masked = jnp.where(pos < lens, x, -jnp.inf)
        bmax = jnp.max(masked, axis=1, keepdims=True)
        cand = jnp.where(masked == bmax, pos, BIG_IDX)
        bidx = jnp.min(cand, axis=1, keepdims=True)

        bidx = jnp.broadcast_to(bidx, (B, 128))
        bmax = jnp.broadcast_to(bmax, (B, 128))
        better = (bmax > rmax) | ((bmax == rmax) & (bidx < ridx))
        rmax = jnp.where(better, bmax, rmax)
        ridx = jnp.where(better, bidx, ridx)

    idx_ref[...] = ridx


@jax.jit
def kernel(stim, vals, lens, Wk, bk, Wv, bv):
    lens2d = lens.astype(jnp.int32).reshape(B, 1)
    x, idx = pl.pallas_call(
        _net_tree_kernel,
        in_specs=[
            pl.BlockSpec((B, H), lambda: (0, 0)),
            pl.BlockSpec(memory_space=pl.ANY),
            pl.BlockSpec((B, 1), lambda: (0, 0)),
            pl.BlockSpec((H, H), lambda: (0, 0)),
            pl.BlockSpec((1, H), lambda: (0, 0)),
            pl.BlockSpec((H, H), lambda: (0, 0)),
            pl.BlockSpec((1, H), lambda: (0, 0)),
        ],
        out_specs=[
            pl.BlockSpec((B, L), lambda: (0, 0)),
            pl.BlockSpec((B, 128), lambda: (0, 0)),
        ],
        out_shape=[
            jax.ShapeDtypeStruct((B, L), jnp.float32),
            jax.ShapeDtypeStruct((B, 128), jnp.int32),
        ],
        scratch_shapes=[
            pltpu.VMEM((NCHUNK, B, LBLK, H), jnp.float32),
            pltpu.SemaphoreType.DMA((NCHUNK,)),
        ],
    )(stim, vals, lens2d, Wk, bk.reshape(1, H), Wv, bv.reshape(1, H))
    return (x, idx[:, 0])
